# Initial kernel scaffold; baseline (speedup 1.0000x reference)
import jax
import jax.numpy as jnp
from jax.experimental import pallas as pl


def kernel(featurein, edge_index, W1_logr, b1_logr, W1_s, b1_s, W4_s, b4_s, Wm_logr, bm_logr, Wm_s, bm_s, Wmu):
    raise NotImplementedError("write your pallas kernel here")


# speedup vs baseline: 79.4549x; 79.4549x over previous
import functools

import jax
import jax.numpy as jnp
from jax import lax
from jax.experimental import pallas as pl
from jax.experimental.pallas import tpu as pltpu
from jax.experimental.pallas import tpu_sc as plsc

N = 10000
B = 16
FEAT = 128
NPAD = 10240
E = 320000
CHUNK = 128
NCORE = 2
NSUB = 16
NW = NCORE * NSUB
CPW = 79
NCHUNK = NW * CPW
EPAD = NCHUNK * CHUNK
RPT = NPAD // NSUB
NBLK = 10
BLK = NPAD // NBLK

@functools.lru_cache(maxsize=None)
def _make_deg():
    mesh = plsc.VectorSubcoreMesh(core_axis_name="c", subcore_axis_name="s")

    @functools.partial(
        pl.kernel,
        out_type=jax.ShapeDtypeStruct((NW, NPAD), jnp.float32),
        mesh=mesh,
        scratch_types=[
            pltpu.VMEM((NPAD,), jnp.float32),
            pltpu.VMEM((CHUNK,), jnp.int32),
        ],
        compiler_params=pltpu.CompilerParams(needs_layout_passes=False),
    )
    def _deg_sc(dst_hbm, zeros_hbm, out_hbm, hist_v, dbuf):
        c = lax.axis_index("c")
        s = lax.axis_index("s")
        wid = c * NSUB + s
        pltpu.sync_copy(zeros_hbm, hist_v)
        ones = jnp.ones((16,), jnp.float32)

        def chunk_body(j, carry):
            ci = wid * CPW + j
            pltpu.sync_copy(dst_hbm.at[ci], dbuf)

            def inner(k, carry2):
                idx = dbuf[pl.ds(k * 16, 16)]
                plsc.addupdate_scatter(hist_v, [idx], ones)
                return carry2

            return lax.fori_loop(0, CHUNK // 16, inner, carry)

        lax.fori_loop(0, CPW, chunk_body, 0)
        pltpu.sync_copy(hist_v, out_hbm.at[wid])

    return _deg_sc


@functools.lru_cache(maxsize=None)
def _make_agg(F):
    mesh = plsc.VectorSubcoreMesh(core_axis_name="c", subcore_axis_name="s")

    @functools.partial(
        pl.kernel,
        out_type=jax.ShapeDtypeStruct((NCORE * NPAD, F), jnp.float32),
        mesh=mesh,
        scratch_types=[
            pltpu.VMEM((CHUNK,), jnp.int32),
            pltpu.VMEM((CHUNK,), jnp.int32),
            pltpu.VMEM((CHUNK, F), jnp.float32),
            pltpu.VMEM_SHARED((NPAD, F), jnp.float32),
            pltpu.SemaphoreType.DMA,
        ],
    )
    def agg(table_hbm, src_hbm, dst_hbm, out_hbm, sbuf, dbuf, rows, acc, sem):
        c = lax.axis_index("c")
        s = lax.axis_index("s")
        wid = c * NSUB + s
        r0 = s * RPT
        pltpu.sync_copy(table_hbm.at[pl.ds(r0, RPT)], acc.at[pl.ds(r0, RPT)])
        plsc.subcore_barrier()

        def chunk_body(j, carry):
            ci = wid * CPW + j
            pltpu.sync_copy(src_hbm.at[ci], sbuf)
            pltpu.sync_copy(dst_hbm.at[ci], dbuf)
            pltpu.async_copy(table_hbm.at[sbuf], rows, sem).wait()
            pltpu.sync_copy(rows, acc.at[dbuf], add=True)
            return carry

        lax.fori_loop(0, CPW, chunk_body, 0)
        plsc.subcore_barrier()
        pltpu.sync_copy(acc.at[pl.ds(r0, RPT)],
                        out_hbm.at[pl.ds(c * NPAD + r0, RPT)])

    return agg


def _run_deg(dsts, zeros1d):
    return _make_deg()(dsts, zeros1d)


def _run_agg128(table, srcs, dsts):
    return _make_agg(128)(table, srcs, dsts)


def _stageA_body(h_ref, x_ref, w_ref, dinv_ref, t1a_ref, t1b_ref):
    deg = jnp.sum(h_ref[...], axis=1, keepdims=True) + 1.0
    dinv = lax.rsqrt(deg)
    dinv_ref[...] = dinv
    t1 = dinv * jnp.dot(x_ref[...], w_ref[...],
                        preferred_element_type=jnp.float32)
    t1a_ref[...] = t1[:, :128]
    t1b_ref[...] = jnp.concatenate(
        [t1[:, 128:], jnp.zeros((t1.shape[0], 112), jnp.float32)], axis=1)


def _stageA(hist_t, xT, Wbd):
    return pl.pallas_call(
        _stageA_body,
        grid=(NBLK,),
        in_specs=[
            pl.BlockSpec((BLK, NW), lambda i: (i, 0)),
            pl.BlockSpec((BLK, 144), lambda i: (i, 0)),
            pl.BlockSpec((144, 144), lambda i: (0, 0)),
        ],
        out_specs=[
            pl.BlockSpec((BLK, 1), lambda i: (i, 0)),
            pl.BlockSpec((BLK, 128), lambda i: (i, 0)),
            pl.BlockSpec((BLK, 128), lambda i: (i, 0)),
        ],
        out_shape=[
            jax.ShapeDtypeStruct((NPAD, 1), jnp.float32),
            jax.ShapeDtypeStruct((NPAD, 128), jnp.float32),
            jax.ShapeDtypeStruct((NPAD, 128), jnp.float32),
        ],
    )(hist_t, xT, Wbd)


def _stageB_body(a0a_ref, a1a_ref, t1a_ref, a0b_ref, a1b_ref, t1b_ref,
                 dinv_ref, w4_ref, b1_ref, o1l_ref, t2_ref):
    dinv = dinv_ref[...]
    fa = a0a_ref[...] + a1a_ref[...] - t1a_ref[...]
    fb = (a0b_ref[...] + a1b_ref[...] - t1b_ref[...])[:, :16]
    o1 = jnp.tanh(dinv * jnp.concatenate([fa, fb], axis=1) + b1_ref[...])
    o1l_ref[...] = o1[:, :48]
    t2 = dinv * jnp.dot(o1[:, 48:], w4_ref[...],
                        preferred_element_type=jnp.float32)
    t2_ref[...] = jnp.concatenate(
        [t2, jnp.zeros((t2.shape[0], 32), jnp.float32)], axis=1)


def _stageB(a0a, a1a, T1a, a0b, a1b, T1b, dinv, Wbd4, bias1):
    blk128 = pl.BlockSpec((BLK, 128), lambda i: (i, 0))
    return pl.pallas_call(
        _stageB_body,
        grid=(NBLK,),
        in_specs=[
            blk128, blk128, blk128, blk128, blk128, blk128,
            pl.BlockSpec((BLK, 1), lambda i: (i, 0)),
            pl.BlockSpec((96, 96), lambda i: (0, 0)),
            pl.BlockSpec((1, 144), lambda i: (0, 0)),
        ],
        out_specs=[
            pl.BlockSpec((BLK, 48), lambda i: (i, 0)),
            pl.BlockSpec((BLK, 128), lambda i: (i, 0)),
        ],
        out_shape=[
            jax.ShapeDtypeStruct((NPAD, 48), jnp.float32),
            jax.ShapeDtypeStruct((NPAD, 128), jnp.float32),
        ],
    )(a0a, a1a, T1a, a0b, a1b, T1b, dinv, Wbd4, bias1)


def _stageC1_body(b0_ref, b1_ref, t2_ref, dinv_ref, b4_ref, o2_ref):
    f = (b0_ref[...] + b1_ref[...] - t2_ref[...])[:, :96]
    o2_ref[...] = jnp.tanh(dinv_ref[...] * f + b4_ref[...])


def _stageC1(b0, b1_, T2, dinv, bias4):
    blk128 = pl.BlockSpec((BLK, 128), lambda i: (i, 0))
    return pl.pallas_call(
        _stageC1_body,
        grid=(NBLK,),
        in_specs=[
            blk128, blk128, blk128,
            pl.BlockSpec((BLK, 1), lambda i: (i, 0)),
            pl.BlockSpec((1, 96), lambda i: (0, 0)),
        ],
        out_specs=pl.BlockSpec((BLK, 96), lambda i: (i, 0)),
        out_shape=jax.ShapeDtypeStruct((NPAD, 96), jnp.float32),
    )(b0, b1_, T2, dinv, bias4)


KL = 3 * NPAD // NBLK
KS = 6 * NPAD // NBLK


def _stageC2_body(wl_ref, xl_ref, ws_ref, xs_ref, bml_ref, bms_ref, wmu_ref,
                  mu_ref, accl, accs):
    i = pl.program_id(0)

    @pl.when(i == 0)
    def _():
        accl[...] = jnp.zeros_like(accl)
        accs[...] = jnp.zeros_like(accs)

    accl[...] += jnp.dot(wl_ref[...], xl_ref[...],
                         preferred_element_type=jnp.float32)
    accs[...] += jnp.dot(ws_ref[...], xs_ref[...],
                         preferred_element_type=jnp.float32)

    @pl.when(i == NBLK - 1)
    def _():
        netl = jnp.tanh(accl[...] + bml_ref[...])
        nets = jnp.tanh(accs[...] + bms_ref[...])
        net_t = jnp.concatenate([netl, nets], axis=0)
        mu_ref[...] = lax.dot_general(
            net_t, wmu_ref[...], (((0,), (1,)), ((), ())),
            preferred_element_type=jnp.float32)


def _stageC2(WmLp, XrL, WmSp, XrS, bmL, bmS, Wmu):
    return pl.pallas_call(
        _stageC2_body,
        grid=(NBLK,),
        in_specs=[
            pl.BlockSpec((FEAT, KL), lambda i: (0, i)),
            pl.BlockSpec((KL, B), lambda i: (i, 0)),
            pl.BlockSpec((FEAT, KS), lambda i: (0, i)),
            pl.BlockSpec((KS, B), lambda i: (i, 0)),
            pl.BlockSpec((FEAT, 1), lambda i: (0, 0)),
            pl.BlockSpec((FEAT, 1), lambda i: (0, 0)),
            pl.BlockSpec((FEAT, 2 * FEAT), lambda i: (0, 0)),
        ],
        out_specs=pl.BlockSpec((B, FEAT), lambda i: (0, 0)),
        out_shape=jax.ShapeDtypeStruct((B, FEAT), jnp.float32),
        scratch_shapes=[
            pltpu.VMEM((FEAT, B), jnp.float32),
            pltpu.VMEM((FEAT, B), jnp.float32),
        ],
    )(WmLp, XrL, WmSp, XrS, bmL, bmS, Wmu)


def kernel(featurein, edge_index, W1_logr, b1_logr, W1_s, b1_s, W4_s, b4_s,
           Wm_logr, bm_logr, Wm_s, bm_s, Wmu):
    f32 = jnp.float32
    I16 = jnp.eye(B, dtype=f32)

    Wbd = jnp.zeros((144, 144), f32)
    Wbd = Wbd.at[:48, :48].set(jnp.kron(W1_logr / 4.0, I16))
    Wbd = Wbd.at[48:, 48:].set(jnp.kron(W1_s / 50.0, I16))
    Wbd4 = jnp.kron(W4_s, I16)
    bias1 = jnp.concatenate([jnp.repeat(b1_logr, B),
                             jnp.repeat(b1_s, B)]).reshape(1, 144)
    bias4 = jnp.repeat(b4_s, B).reshape(1, 96)

    xT = jnp.transpose(featurein, (1, 2, 0)).reshape(N, 9 * B)
    xT = jnp.pad(xT, ((0, NPAD - N), (0, 0)))

    pad_idx = jnp.full((EPAD - E,), NPAD - 1, jnp.int32)
    srcs = jnp.concatenate([edge_index[0], pad_idx]).reshape(NCHUNK, CHUNK)
    dsts = jnp.concatenate([edge_index[1], pad_idx]).reshape(NCHUNK, CHUNK)

    zeros1d = jnp.zeros((NPAD,), f32)
    hist = _run_deg(dsts, zeros1d)
    hist_t = hist.T

    dinv, T1a, T1b = _stageA(hist_t, xT, Wbd)

    acc1a = _run_agg128(T1a, srcs, dsts)
    acc1b = _run_agg128(T1b, srcs, dsts)
    o1logr, T2 = _stageB(acc1a[:NPAD], acc1a[NPAD:], T1a,
                         acc1b[:NPAD], acc1b[NPAD:], T1b,
                         dinv, Wbd4, bias1)

    acc2 = _run_agg128(T2, srcs, dsts)
    o2 = _stageC1(acc2[:NPAD], acc2[NPAD:], T2, dinv, bias4)

    XrL = o1logr.reshape(3 * NPAD, B)
    XrS = o2.reshape(6 * NPAD, B)
    WmLp = jnp.pad(Wm_logr, ((0, 0), (0, 3 * NPAD - 3 * N)))
    WmSp = jnp.pad(Wm_s, ((0, 0), (0, 6 * NPAD - 6 * N)))

    mu = _stageC2(WmLp, XrL, WmSp, XrS,
                  bm_logr.reshape(FEAT, 1), bm_s.reshape(FEAT, 1), Wmu)
    return mu


# speedup vs baseline: 104.5631x; 1.3160x over previous
import functools

import jax
import jax.numpy as jnp
from jax import lax
from jax.experimental import pallas as pl
from jax.experimental.pallas import tpu as pltpu
from jax.experimental.pallas import tpu_sc as plsc

N = 10000
B = 16
FEAT = 128
NPAD = 10240
E = 320000
CHUNK = 128
NCORE = 2
NSUB = 16
NW = NCORE * NSUB
CPW = 79
NCHUNK = NW * CPW
EPAD = NCHUNK * CHUNK
RPT = NPAD // NSUB
NBLK = 10
BLK = NPAD // NBLK

@functools.lru_cache(maxsize=None)
def _make_deg():
    mesh = plsc.VectorSubcoreMesh(core_axis_name="c", subcore_axis_name="s")

    @functools.partial(
        pl.kernel,
        out_type=jax.ShapeDtypeStruct((NW, NPAD), jnp.float32),
        mesh=mesh,
        scratch_types=[
            pltpu.VMEM((NPAD,), jnp.float32),
            pltpu.VMEM((CHUNK,), jnp.int32),
        ],
        compiler_params=pltpu.CompilerParams(needs_layout_passes=False),
    )
    def _deg_sc(dst_hbm, zeros_hbm, out_hbm, hist_v, dbuf):
        c = lax.axis_index("c")
        s = lax.axis_index("s")
        wid = c * NSUB + s
        pltpu.sync_copy(zeros_hbm, hist_v)
        ones = jnp.ones((16,), jnp.float32)

        def chunk_body(j, carry):
            ci = wid * CPW + j
            pltpu.sync_copy(dst_hbm.at[ci], dbuf)

            def inner(k, carry2):
                idx = dbuf[pl.ds(k * 16, 16)]
                plsc.addupdate_scatter(hist_v, [idx], ones)
                return carry2

            return lax.fori_loop(0, CHUNK // 16, inner, carry)

        lax.fori_loop(0, CPW, chunk_body, 0)
        pltpu.sync_copy(hist_v, out_hbm.at[wid])

    return _deg_sc


@functools.lru_cache(maxsize=None)
def _make_agg(F):
    assert CPW % 2 == 1
    mesh = plsc.VectorSubcoreMesh(core_axis_name="c", subcore_axis_name="s")

    @functools.partial(
        pl.kernel,
        out_type=[jax.ShapeDtypeStruct((NPAD, F), jnp.float32)] * NCORE,
        mesh=mesh,
        scratch_types=[
            pltpu.VMEM((CHUNK,), jnp.int32),
            pltpu.VMEM((CHUNK,), jnp.int32),
            pltpu.VMEM((CHUNK, F), jnp.float32),
            pltpu.VMEM((CHUNK,), jnp.int32),
            pltpu.VMEM((CHUNK,), jnp.int32),
            pltpu.VMEM((CHUNK, F), jnp.float32),
            pltpu.VMEM_SHARED((NPAD, F), jnp.float32),
            pltpu.SemaphoreType.DMA,
            pltpu.SemaphoreType.DMA,
        ],
    )
    def agg(table_hbm, src_hbm, dst_hbm, out0_hbm, out1_hbm,
            sb0, db0, rw0, sb1, db1, rw1, acc, sem0, sem1):
        c = lax.axis_index("c")
        s = lax.axis_index("s")
        wid = c * NSUB + s
        row0 = s * RPT
        pltpu.sync_copy(table_hbm.at[pl.ds(row0, RPT)],
                        acc.at[pl.ds(row0, RPT)])
        plsc.subcore_barrier()

        base = wid * CPW
        dummy = table_hbm.at[pl.ds(0, CHUNK)]

        pltpu.sync_copy(src_hbm.at[base], sb0)
        pltpu.sync_copy(dst_hbm.at[base], db0)
        pltpu.async_copy(table_hbm.at[sb0], rw0, sem0)

        def pair_body(j, carry):
            c0 = base + 2 * j
            pltpu.sync_copy(src_hbm.at[c0 + 1], sb1)
            pltpu.sync_copy(dst_hbm.at[c0 + 1], db1)
            pltpu.async_copy(table_hbm.at[sb1], rw1, sem1)
            pltpu.make_async_copy(dummy, rw0, sem0).wait()
            pltpu.sync_copy(rw0, acc.at[db0], add=True)
            pltpu.sync_copy(src_hbm.at[c0 + 2], sb0)
            pltpu.sync_copy(dst_hbm.at[c0 + 2], db0)
            pltpu.async_copy(table_hbm.at[sb0], rw0, sem0)
            pltpu.make_async_copy(dummy, rw1, sem1).wait()
            pltpu.sync_copy(rw1, acc.at[db1], add=True)
            return carry

        lax.fori_loop(0, (CPW - 1) // 2, pair_body, 0)
        pltpu.make_async_copy(dummy, rw0, sem0).wait()
        pltpu.sync_copy(rw0, acc.at[db0], add=True)

        plsc.subcore_barrier()

        @pl.when(c == 0)
        def _():
            pltpu.sync_copy(acc.at[pl.ds(row0, RPT)],
                            out0_hbm.at[pl.ds(row0, RPT)])

        @pl.when(c == 1)
        def _():
            pltpu.sync_copy(acc.at[pl.ds(row0, RPT)],
                            out1_hbm.at[pl.ds(row0, RPT)])

    return agg


def _run_deg(dsts, zeros1d):
    return _make_deg()(dsts, zeros1d)


def _run_agg128(table, srcs, dsts):
    return _make_agg(128)(table, srcs, dsts)


def _stageA_body(h_ref, x_ref, w_ref, dinv_ref, t1a_ref, t1b_ref):
    deg = jnp.sum(h_ref[...], axis=1, keepdims=True) + 1.0
    dinv = lax.rsqrt(deg)
    dinv_ref[...] = dinv
    t1 = dinv * jnp.dot(x_ref[...], w_ref[...],
                        preferred_element_type=jnp.float32)
    t1a_ref[...] = t1[:, :128]
    t1b_ref[...] = jnp.concatenate(
        [t1[:, 128:], jnp.zeros((t1.shape[0], 112), jnp.float32)], axis=1)


def _stageA(hist_t, xT, Wbd):
    return pl.pallas_call(
        _stageA_body,
        grid=(NBLK,),
        in_specs=[
            pl.BlockSpec((BLK, NW), lambda i: (i, 0)),
            pl.BlockSpec((BLK, 144), lambda i: (i, 0)),
            pl.BlockSpec((144, 144), lambda i: (0, 0)),
        ],
        out_specs=[
            pl.BlockSpec((BLK, 1), lambda i: (i, 0)),
            pl.BlockSpec((BLK, 128), lambda i: (i, 0)),
            pl.BlockSpec((BLK, 128), lambda i: (i, 0)),
        ],
        out_shape=[
            jax.ShapeDtypeStruct((NPAD, 1), jnp.float32),
            jax.ShapeDtypeStruct((NPAD, 128), jnp.float32),
            jax.ShapeDtypeStruct((NPAD, 128), jnp.float32),
        ],
    )(hist_t, xT, Wbd)


def _stageB_body(a0a_ref, a1a_ref, t1a_ref, a0b_ref, a1b_ref, t1b_ref,
                 dinv_ref, w4_ref, b1_ref, o1l_ref, t2_ref):
    dinv = dinv_ref[...]
    fa = a0a_ref[...] + a1a_ref[...] - t1a_ref[...]
    fb = (a0b_ref[...] + a1b_ref[...] - t1b_ref[...])[:, :16]
    o1 = jnp.tanh(dinv * jnp.concatenate([fa, fb], axis=1) + b1_ref[...])
    o1l_ref[...] = o1[:, :48]
    t2 = dinv * jnp.dot(o1[:, 48:], w4_ref[...],
                        preferred_element_type=jnp.float32)
    t2_ref[...] = jnp.concatenate(
        [t2, jnp.zeros((t2.shape[0], 32), jnp.float32)], axis=1)


def _stageB(a0a, a1a, T1a, a0b, a1b, T1b, dinv, Wbd4, bias1):
    blk128 = pl.BlockSpec((BLK, 128), lambda i: (i, 0))
    return pl.pallas_call(
        _stageB_body,
        grid=(NBLK,),
        in_specs=[
            blk128, blk128, blk128, blk128, blk128, blk128,
            pl.BlockSpec((BLK, 1), lambda i: (i, 0)),
            pl.BlockSpec((96, 96), lambda i: (0, 0)),
            pl.BlockSpec((1, 144), lambda i: (0, 0)),
        ],
        out_specs=[
            pl.BlockSpec((BLK, 48), lambda i: (i, 0)),
            pl.BlockSpec((BLK, 128), lambda i: (i, 0)),
        ],
        out_shape=[
            jax.ShapeDtypeStruct((NPAD, 48), jnp.float32),
            jax.ShapeDtypeStruct((NPAD, 128), jnp.float32),
        ],
    )(a0a, a1a, T1a, a0b, a1b, T1b, dinv, Wbd4, bias1)


def _stageC1_body(b0_ref, b1_ref, t2_ref, dinv_ref, b4_ref, o2_ref):
    f = (b0_ref[...] + b1_ref[...] - t2_ref[...])[:, :96]
    o2_ref[...] = jnp.tanh(dinv_ref[...] * f + b4_ref[...])


def _stageC1(b0, b1_, T2, dinv, bias4):
    blk128 = pl.BlockSpec((BLK, 128), lambda i: (i, 0))
    return pl.pallas_call(
        _stageC1_body,
        grid=(NBLK,),
        in_specs=[
            blk128, blk128, blk128,
            pl.BlockSpec((BLK, 1), lambda i: (i, 0)),
            pl.BlockSpec((1, 96), lambda i: (0, 0)),
        ],
        out_specs=pl.BlockSpec((BLK, 96), lambda i: (i, 0)),
        out_shape=jax.ShapeDtypeStruct((NPAD, 96), jnp.float32),
    )(b0, b1_, T2, dinv, bias4)


KL = 3 * NPAD // NBLK
KS = 6 * NPAD // NBLK


def _stageC2_body(wl_ref, xl_ref, ws_ref, xs_ref, bml_ref, bms_ref, wmu_ref,
                  mu_ref, accl, accs):
    i = pl.program_id(0)

    @pl.when(i == 0)
    def _():
        accl[...] = jnp.zeros_like(accl)
        accs[...] = jnp.zeros_like(accs)

    accl[...] += jnp.dot(wl_ref[...], xl_ref[...],
                         preferred_element_type=jnp.float32)
    accs[...] += jnp.dot(ws_ref[...], xs_ref[...],
                         preferred_element_type=jnp.float32)

    @pl.when(i == NBLK - 1)
    def _():
        netl = jnp.tanh(accl[...] + bml_ref[...])
        nets = jnp.tanh(accs[...] + bms_ref[...])
        net_t = jnp.concatenate([netl, nets], axis=0)
        mu_ref[...] = lax.dot_general(
            net_t, wmu_ref[...], (((0,), (1,)), ((), ())),
            preferred_element_type=jnp.float32)


def _stageC2(WmLp, XrL, WmSp, XrS, bmL, bmS, Wmu):
    return pl.pallas_call(
        _stageC2_body,
        grid=(NBLK,),
        in_specs=[
            pl.BlockSpec((FEAT, KL), lambda i: (0, i)),
            pl.BlockSpec((KL, B), lambda i: (i, 0)),
            pl.BlockSpec((FEAT, KS), lambda i: (0, i)),
            pl.BlockSpec((KS, B), lambda i: (i, 0)),
            pl.BlockSpec((FEAT, 1), lambda i: (0, 0)),
            pl.BlockSpec((FEAT, 1), lambda i: (0, 0)),
            pl.BlockSpec((FEAT, 2 * FEAT), lambda i: (0, 0)),
        ],
        out_specs=pl.BlockSpec((B, FEAT), lambda i: (0, 0)),
        out_shape=jax.ShapeDtypeStruct((B, FEAT), jnp.float32),
        scratch_shapes=[
            pltpu.VMEM((FEAT, B), jnp.float32),
            pltpu.VMEM((FEAT, B), jnp.float32),
        ],
    )(WmLp, XrL, WmSp, XrS, bmL, bmS, Wmu)


def kernel(featurein, edge_index, W1_logr, b1_logr, W1_s, b1_s, W4_s, b4_s,
           Wm_logr, bm_logr, Wm_s, bm_s, Wmu):
    f32 = jnp.float32
    I16 = jnp.eye(B, dtype=f32)

    Wbd = jnp.zeros((144, 144), f32)
    Wbd = Wbd.at[:48, :48].set(jnp.kron(W1_logr / 4.0, I16))
    Wbd = Wbd.at[48:, 48:].set(jnp.kron(W1_s / 50.0, I16))
    Wbd4 = jnp.kron(W4_s, I16)
    bias1 = jnp.concatenate([jnp.repeat(b1_logr, B),
                             jnp.repeat(b1_s, B)]).reshape(1, 144)
    bias4 = jnp.repeat(b4_s, B).reshape(1, 96)

    xT = jnp.transpose(featurein, (1, 2, 0)).reshape(N, 9 * B)
    xT = jnp.pad(xT, ((0, NPAD - N), (0, 0)))

    pad_idx = jnp.full((EPAD - E,), NPAD - 1, jnp.int32)
    srcs = jnp.concatenate([edge_index[0], pad_idx]).reshape(NCHUNK, CHUNK)
    dsts = jnp.concatenate([edge_index[1], pad_idx]).reshape(NCHUNK, CHUNK)

    zeros1d = jnp.zeros((NPAD,), f32)
    hist = _run_deg(dsts, zeros1d)
    hist_t = hist.T

    dinv, T1a, T1b = _stageA(hist_t, xT, Wbd)

    a0a, a1a = _run_agg128(T1a, srcs, dsts)
    a0b, a1b = _run_agg128(T1b, srcs, dsts)
    o1logr, T2 = _stageB(a0a, a1a, T1a, a0b, a1b, T1b, dinv, Wbd4, bias1)

    b0, b1_ = _run_agg128(T2, srcs, dsts)
    o2 = _stageC1(b0, b1_, T2, dinv, bias4)

    XrL = o1logr.reshape(3 * NPAD, B)
    XrS = o2.reshape(6 * NPAD, B)
    WmLp = jnp.pad(Wm_logr, ((0, 0), (0, 3 * NPAD - 3 * N)))
    WmSp = jnp.pad(Wm_s, ((0, 0), (0, 6 * NPAD - 6 * N)))

    mu = _stageC2(WmLp, XrL, WmSp, XrS,
                  bm_logr.reshape(FEAT, 1), bm_s.reshape(FEAT, 1), Wmu)
    return mu


# speedup vs baseline: 219.2854x; 2.0972x over previous
import functools

import jax
import jax.numpy as jnp
from jax import lax
from jax.experimental import pallas as pl
from jax.experimental.pallas import tpu as pltpu
from jax.experimental.pallas import tpu_sc as plsc

N = 10000
B = 16
FEAT = 128
NPAD = 10240
E = 320000
CHUNK = 128
NCORE = 2
NSUB = 16
NW = NCORE * NSUB
CPW = 80
NCHUNK = NW * CPW
EPAD = NCHUNK * CHUNK
RPT = NPAD // NSUB
NBLK = 10
BLK = NPAD // NBLK

@functools.lru_cache(maxsize=None)
def _make_deg():
    mesh = plsc.VectorSubcoreMesh(core_axis_name="c", subcore_axis_name="s")

    @functools.partial(
        pl.kernel,
        out_type=jax.ShapeDtypeStruct((NW, NPAD), jnp.float32),
        mesh=mesh,
        scratch_types=[
            pltpu.VMEM((NPAD,), jnp.float32),
            pltpu.VMEM((CPW, CHUNK), jnp.int32),
        ],
        compiler_params=pltpu.CompilerParams(needs_layout_passes=False),
    )
    def _deg_sc(dst_hbm, zeros_hbm, out_hbm, hist_v, dbig):
        c = lax.axis_index("c")
        s = lax.axis_index("s")
        wid = c * NSUB + s
        pltpu.sync_copy(zeros_hbm, hist_v)
        pltpu.sync_copy(dst_hbm.at[pl.ds(wid * CPW, CPW)], dbig)
        ones = jnp.ones((16,), jnp.float32)

        def chunk_body(j, carry):
            def inner(k, carry2):
                idx = dbig[j, pl.ds(k * 16, 16)]
                plsc.addupdate_scatter(hist_v, [idx], ones)
                return carry2

            return lax.fori_loop(0, CHUNK // 16, inner, carry)

        lax.fori_loop(0, CPW, chunk_body, 0)
        pltpu.sync_copy(hist_v, out_hbm.at[wid])

    return _deg_sc


@functools.lru_cache(maxsize=None)
def _make_agg(F):
    assert CPW % 2 == 0
    mesh = plsc.VectorSubcoreMesh(core_axis_name="c", subcore_axis_name="s")

    @functools.partial(
        pl.kernel,
        out_type=[jax.ShapeDtypeStruct((NPAD, F), jnp.float32)] * NCORE,
        mesh=mesh,
        scratch_types=[
            pltpu.VMEM((CPW // 2, CHUNK), jnp.int32),
            pltpu.VMEM((CPW // 2, CHUNK), jnp.int32),
            pltpu.VMEM((CHUNK, F), jnp.float32),
            pltpu.VMEM((CHUNK, F), jnp.float32),
            pltpu.VMEM_SHARED((NPAD, F), jnp.float32),
            pltpu.SemaphoreType.DMA,
            pltpu.SemaphoreType.DMA,
        ],
    )
    def agg(table_hbm, src_hbm, dst_hbm, out0_hbm, out1_hbm,
            sbig, dbig, rw0, rw1, acc, sem0, sem1):
        c = lax.axis_index("c")
        s = lax.axis_index("s")
        wid = c * NSUB + s
        row0 = s * RPT
        pltpu.sync_copy(table_hbm.at[pl.ds(row0, RPT)],
                        acc.at[pl.ds(row0, RPT)])
        plsc.subcore_barrier()

        HCPW = CPW // 2
        dummy = table_hbm.at[pl.ds(0, CHUNK)]

        for h in range(2):
            pltpu.sync_copy(src_hbm.at[pl.ds(wid * CPW + h * HCPW, HCPW)],
                            sbig)
            pltpu.sync_copy(dst_hbm.at[pl.ds(wid * CPW + h * HCPW, HCPW)],
                            dbig)
            pltpu.async_copy(table_hbm.at[sbig.at[0]], rw0, sem0)

            def pair_body(j, carry):
                c0 = 2 * j
                pltpu.async_copy(table_hbm.at[sbig.at[c0 + 1]], rw1, sem1)
                pltpu.make_async_copy(dummy, rw0, sem0).wait()
                pltpu.sync_copy(rw0, acc.at[dbig.at[c0]], add=True)
                pltpu.async_copy(table_hbm.at[sbig.at[c0 + 2]], rw0, sem0)
                pltpu.make_async_copy(dummy, rw1, sem1).wait()
                pltpu.sync_copy(rw1, acc.at[dbig.at[c0 + 1]], add=True)
                return carry

            lax.fori_loop(0, (HCPW - 2) // 2, pair_body, 0)
            pltpu.async_copy(table_hbm.at[sbig.at[HCPW - 1]], rw1, sem1)
            pltpu.make_async_copy(dummy, rw0, sem0).wait()
            pltpu.sync_copy(rw0, acc.at[dbig.at[HCPW - 2]], add=True)
            pltpu.make_async_copy(dummy, rw1, sem1).wait()
            pltpu.sync_copy(rw1, acc.at[dbig.at[HCPW - 1]], add=True)

        plsc.subcore_barrier()

        @pl.when(c == 0)
        def _():
            pltpu.sync_copy(acc.at[pl.ds(row0, RPT)],
                            out0_hbm.at[pl.ds(row0, RPT)])

        @pl.when(c == 1)
        def _():
            pltpu.sync_copy(acc.at[pl.ds(row0, RPT)],
                            out1_hbm.at[pl.ds(row0, RPT)])

    return agg


def _run_deg(dsts, zeros1d):
    return _make_deg()(dsts, zeros1d)


def _run_agg128(table, srcs, dsts):
    return _make_agg(128)(table, srcs, dsts)


def _stageA_body(h_ref, x_ref, w_ref, dinv_ref, t1a_ref, t1b_ref):
    deg = jnp.sum(h_ref[...], axis=1, keepdims=True) + 1.0
    dinv = lax.rsqrt(deg)
    dinv_ref[...] = dinv
    t1 = dinv * jnp.dot(x_ref[...], w_ref[...],
                        preferred_element_type=jnp.float32)
    t1a_ref[...] = t1[:, :128]
    t1b_ref[...] = jnp.concatenate(
        [t1[:, 128:], jnp.zeros((t1.shape[0], 112), jnp.float32)], axis=1)


def _stageA(hist_t, xT, Wbd):
    return pl.pallas_call(
        _stageA_body,
        grid=(NBLK,),
        in_specs=[
            pl.BlockSpec((BLK, NW), lambda i: (i, 0)),
            pl.BlockSpec((BLK, 144), lambda i: (i, 0)),
            pl.BlockSpec((144, 144), lambda i: (0, 0)),
        ],
        out_specs=[
            pl.BlockSpec((BLK, 1), lambda i: (i, 0)),
            pl.BlockSpec((BLK, 128), lambda i: (i, 0)),
            pl.BlockSpec((BLK, 128), lambda i: (i, 0)),
        ],
        out_shape=[
            jax.ShapeDtypeStruct((NPAD, 1), jnp.float32),
            jax.ShapeDtypeStruct((NPAD, 128), jnp.float32),
            jax.ShapeDtypeStruct((NPAD, 128), jnp.float32),
        ],
    )(hist_t, xT, Wbd)


def _stageB_body(a0a_ref, a1a_ref, t1a_ref, a0b_ref, a1b_ref, t1b_ref,
                 dinv_ref, w4_ref, b1_ref, o1l_ref, t2_ref):
    dinv = dinv_ref[...]
    fa = a0a_ref[...] + a1a_ref[...] - t1a_ref[...]
    fb = (a0b_ref[...] + a1b_ref[...] - t1b_ref[...])[:, :16]
    o1 = jnp.tanh(dinv * jnp.concatenate([fa, fb], axis=1) + b1_ref[...])
    o1l_ref[...] = o1[:, :48]
    t2 = dinv * jnp.dot(o1[:, 48:], w4_ref[...],
                        preferred_element_type=jnp.float32)
    t2_ref[...] = jnp.concatenate(
        [t2, jnp.zeros((t2.shape[0], 32), jnp.float32)], axis=1)


def _stageB(a0a, a1a, T1a, a0b, a1b, T1b, dinv, Wbd4, bias1):
    blk128 = pl.BlockSpec((BLK, 128), lambda i: (i, 0))
    return pl.pallas_call(
        _stageB_body,
        grid=(NBLK,),
        in_specs=[
            blk128, blk128, blk128, blk128, blk128, blk128,
            pl.BlockSpec((BLK, 1), lambda i: (i, 0)),
            pl.BlockSpec((96, 96), lambda i: (0, 0)),
            pl.BlockSpec((1, 144), lambda i: (0, 0)),
        ],
        out_specs=[
            pl.BlockSpec((BLK, 48), lambda i: (i, 0)),
            pl.BlockSpec((BLK, 128), lambda i: (i, 0)),
        ],
        out_shape=[
            jax.ShapeDtypeStruct((NPAD, 48), jnp.float32),
            jax.ShapeDtypeStruct((NPAD, 128), jnp.float32),
        ],
    )(a0a, a1a, T1a, a0b, a1b, T1b, dinv, Wbd4, bias1)


def _stageC1_body(b0_ref, b1_ref, t2_ref, dinv_ref, b4_ref, o2_ref):
    f = (b0_ref[...] + b1_ref[...] - t2_ref[...])[:, :96]
    o2_ref[...] = jnp.tanh(dinv_ref[...] * f + b4_ref[...])


def _stageC1(b0, b1_, T2, dinv, bias4):
    blk128 = pl.BlockSpec((BLK, 128), lambda i: (i, 0))
    return pl.pallas_call(
        _stageC1_body,
        grid=(NBLK,),
        in_specs=[
            blk128, blk128, blk128,
            pl.BlockSpec((BLK, 1), lambda i: (i, 0)),
            pl.BlockSpec((1, 96), lambda i: (0, 0)),
        ],
        out_specs=pl.BlockSpec((BLK, 96), lambda i: (i, 0)),
        out_shape=jax.ShapeDtypeStruct((NPAD, 96), jnp.float32),
    )(b0, b1_, T2, dinv, bias4)


NBLK2 = 26
KL = 29952 // NBLK2
KS = 59904 // NBLK2


def _stageC2_body(wl_ref, xl_ref, ws_ref, xs_ref, wlt_ref, xlt_ref,
                  wst_ref, xst_ref, bml_ref, bms_ref, wmu_ref,
                  mu_ref, accl, accs):
    i = pl.program_id(0)

    @pl.when(i == 0)
    def _():
        accl[...] = jnp.zeros_like(accl)
        accs[...] = jnp.zeros_like(accs)

    accl[...] += jnp.dot(wl_ref[...], xl_ref[...],
                         preferred_element_type=jnp.float32)
    accs[...] += jnp.dot(ws_ref[...], xs_ref[...],
                         preferred_element_type=jnp.float32)

    @pl.when(i == NBLK2 - 1)
    def _():
        tl = accl[...] + jnp.dot(wlt_ref[...], xlt_ref[...],
                                 preferred_element_type=jnp.float32)
        ts = accs[...] + jnp.dot(wst_ref[...], xst_ref[...],
                                 preferred_element_type=jnp.float32)
        netl = jnp.tanh(tl + bml_ref[...])
        nets = jnp.tanh(ts + bms_ref[...])
        net_t = jnp.concatenate([netl, nets], axis=0)
        mu_ref[...] = lax.dot_general(
            net_t, wmu_ref[...], (((0,), (1,)), ((), ())),
            preferred_element_type=jnp.float32)


def _stageC2(WmL, XrL, WmS, XrS, WmLt, WmSt, bmL, bmS, Wmu):
    return pl.pallas_call(
        _stageC2_body,
        grid=(NBLK2,),
        in_specs=[
            pl.BlockSpec((FEAT, KL), lambda i: (0, i)),
            pl.BlockSpec((KL, B), lambda i: (i, 0)),
            pl.BlockSpec((FEAT, KS), lambda i: (0, i)),
            pl.BlockSpec((KS, B), lambda i: (i, 0)),
            pl.BlockSpec((FEAT, 128), lambda i: (0, 0)),
            pl.BlockSpec((128, B), lambda i: (234, 0)),
            pl.BlockSpec((FEAT, 128), lambda i: (0, 0)),
            pl.BlockSpec((128, B), lambda i: (468, 0)),
            pl.BlockSpec((FEAT, 1), lambda i: (0, 0)),
            pl.BlockSpec((FEAT, 1), lambda i: (0, 0)),
            pl.BlockSpec((FEAT, 2 * FEAT), lambda i: (0, 0)),
        ],
        out_specs=pl.BlockSpec((B, FEAT), lambda i: (0, 0)),
        out_shape=jax.ShapeDtypeStruct((B, FEAT), jnp.float32),
        scratch_shapes=[
            pltpu.VMEM((FEAT, B), jnp.float32),
            pltpu.VMEM((FEAT, B), jnp.float32),
        ],
    )(WmL, XrL, WmS, XrS, WmLt, XrL, WmSt, XrS, bmL, bmS, Wmu)


def kernel(featurein, edge_index, W1_logr, b1_logr, W1_s, b1_s, W4_s, b4_s,
           Wm_logr, bm_logr, Wm_s, bm_s, Wmu):
    f32 = jnp.float32
    I16 = jnp.eye(B, dtype=f32)

    Wbd = jnp.zeros((144, 144), f32)
    Wbd = Wbd.at[:48, :48].set(jnp.kron(W1_logr / 4.0, I16))
    Wbd = Wbd.at[48:, 48:].set(jnp.kron(W1_s / 50.0, I16))
    Wbd4 = jnp.kron(W4_s, I16)
    bias1 = jnp.concatenate([jnp.repeat(b1_logr, B),
                             jnp.repeat(b1_s, B)]).reshape(1, 144)
    bias4 = jnp.repeat(b4_s, B).reshape(1, 96)

    xT = jnp.transpose(featurein, (1, 2, 0)).reshape(N, 9 * B)
    xT = jnp.pad(xT, ((0, NPAD - N), (0, 0)))

    pad_idx = N + jnp.arange(EPAD - E, dtype=jnp.int32) % (NPAD - N)
    srcs = jnp.concatenate([edge_index[0], pad_idx]).reshape(NCHUNK, CHUNK)
    dsts = jnp.concatenate([edge_index[1], pad_idx]).reshape(NCHUNK, CHUNK)

    zeros1d = jnp.zeros((NPAD,), f32)
    hist = _run_deg(dsts, zeros1d)
    hist_t = hist.T

    dinv, T1a, T1b = _stageA(hist_t, xT, Wbd)

    a0a, a1a = _run_agg128(T1a, srcs, dsts)
    a0b, a1b = _run_agg128(T1b, srcs, dsts)
    o1logr, T2 = _stageB(a0a, a1a, T1a, a0b, a1b, T1b, dinv, Wbd4, bias1)

    b0, b1_ = _run_agg128(T2, srcs, dsts)
    o2 = _stageC1(b0, b1_, T2, dinv, bias4)

    XrL = o1logr.reshape(3 * NPAD, B)
    XrS = o2.reshape(6 * NPAD, B)
    WmLt = jnp.pad(Wm_logr[:, NBLK2 * KL:], ((0, 0), (0, 128 - 48)))
    WmSt = jnp.pad(Wm_s[:, NBLK2 * KS:], ((0, 0), (0, 128 - 96)))

    mu = _stageC2(Wm_logr, XrL, Wm_s, XrS, WmLt, WmSt,
                  bm_logr.reshape(FEAT, 1), bm_s.reshape(FEAT, 1), Wmu)
    return mu


# speedup vs baseline: 233.3941x; 1.0643x over previous
import functools

import jax
import jax.numpy as jnp
from jax import lax
from jax.experimental import pallas as pl
from jax.experimental.pallas import tpu as pltpu
from jax.experimental.pallas import tpu_sc as plsc

N = 10000
B = 16
FEAT = 128
NPAD = 10240
E = 320000
CHUNK = 128
NCORE = 2
NSUB = 16
NW = NCORE * NSUB
CPW = 80
NCHUNK = NW * CPW
EPAD = NCHUNK * CHUNK
RPT = NPAD // NSUB
NBLK = 10
BLK = NPAD // NBLK

@functools.lru_cache(maxsize=None)
def _make_deg():
    mesh = plsc.VectorSubcoreMesh(core_axis_name="c", subcore_axis_name="s")

    @functools.partial(
        pl.kernel,
        out_type=jax.ShapeDtypeStruct((NW, NPAD), jnp.float32),
        mesh=mesh,
        scratch_types=[
            pltpu.VMEM((NPAD,), jnp.float32),
            pltpu.VMEM((CPW, CHUNK), jnp.int32),
        ],
        compiler_params=pltpu.CompilerParams(needs_layout_passes=False),
    )
    def _deg_sc(dst_hbm, zeros_hbm, out_hbm, hist_v, dbig):
        c = lax.axis_index("c")
        s = lax.axis_index("s")
        wid = c * NSUB + s
        pltpu.sync_copy(zeros_hbm, hist_v)
        pltpu.sync_copy(dst_hbm.at[pl.ds(wid * CPW, CPW)], dbig)
        ones = jnp.ones((16,), jnp.float32)

        def chunk_body(j, carry):
            def inner(k, carry2):
                idx = dbig[j, pl.ds(k * 16, 16)]
                plsc.addupdate_scatter(hist_v, [idx], ones)
                return carry2

            return lax.fori_loop(0, CHUNK // 16, inner, carry)

        lax.fori_loop(0, CPW, chunk_body, 0)
        pltpu.sync_copy(hist_v, out_hbm.at[wid])

    return _deg_sc


@functools.lru_cache(maxsize=None)
def _make_agg(F):
    assert CPW % 2 == 0
    mesh = plsc.VectorSubcoreMesh(core_axis_name="c", subcore_axis_name="s")

    @functools.partial(
        pl.kernel,
        out_type=[jax.ShapeDtypeStruct((NPAD, F), jnp.float32)] * NCORE,
        mesh=mesh,
        scratch_types=[
            pltpu.VMEM((CPW // 2, CHUNK), jnp.int32),
            pltpu.VMEM((CPW // 2, CHUNK), jnp.int32),
            pltpu.VMEM((CHUNK, F), jnp.float32),
            pltpu.VMEM((CHUNK, F), jnp.float32),
            pltpu.VMEM_SHARED((NPAD, F), jnp.float32),
            pltpu.SemaphoreType.DMA,
            pltpu.SemaphoreType.DMA,
        ],
    )
    def agg(table_hbm, src_hbm, dst_hbm, out0_hbm, out1_hbm,
            sbig, dbig, rw0, rw1, acc, sem0, sem1):
        c = lax.axis_index("c")
        s = lax.axis_index("s")
        wid = c * NSUB + s
        row0 = s * RPT
        pltpu.sync_copy(table_hbm.at[pl.ds(row0, RPT)],
                        acc.at[pl.ds(row0, RPT)])
        plsc.subcore_barrier()

        HCPW = CPW // 2
        dummy = table_hbm.at[pl.ds(0, CHUNK)]

        for h in range(2):
            pltpu.sync_copy(src_hbm.at[pl.ds(wid * CPW + h * HCPW, HCPW)],
                            sbig)
            pltpu.sync_copy(dst_hbm.at[pl.ds(wid * CPW + h * HCPW, HCPW)],
                            dbig)
            pltpu.async_copy(table_hbm.at[sbig.at[0]], rw0, sem0)

            def pair_body(j, carry):
                c0 = 2 * j
                pltpu.async_copy(table_hbm.at[sbig.at[c0 + 1]], rw1, sem1)
                pltpu.make_async_copy(dummy, rw0, sem0).wait()
                pltpu.sync_copy(rw0, acc.at[dbig.at[c0]], add=True)
                pltpu.async_copy(table_hbm.at[sbig.at[c0 + 2]], rw0, sem0)
                pltpu.make_async_copy(dummy, rw1, sem1).wait()
                pltpu.sync_copy(rw1, acc.at[dbig.at[c0 + 1]], add=True)
                return carry

            lax.fori_loop(0, (HCPW - 2) // 2, pair_body, 0)
            pltpu.async_copy(table_hbm.at[sbig.at[HCPW - 1]], rw1, sem1)
            pltpu.make_async_copy(dummy, rw0, sem0).wait()
            pltpu.sync_copy(rw0, acc.at[dbig.at[HCPW - 2]], add=True)
            pltpu.make_async_copy(dummy, rw1, sem1).wait()
            pltpu.sync_copy(rw1, acc.at[dbig.at[HCPW - 1]], add=True)

        plsc.subcore_barrier()

        @pl.when(c == 0)
        def _():
            pltpu.sync_copy(acc.at[pl.ds(row0, RPT)],
                            out0_hbm.at[pl.ds(row0, RPT)])

        @pl.when(c == 1)
        def _():
            pltpu.sync_copy(acc.at[pl.ds(row0, RPT)],
                            out1_hbm.at[pl.ds(row0, RPT)])

    return agg


def _run_deg(dsts, zeros1d):
    return _make_deg()(dsts, zeros1d)


def _run_agg128(table, srcs, dsts):
    return _make_agg(128)(table, srcs, dsts)


def _stageA_body(h_ref, x_ref, w_ref, dinv_ref, t1a_ref, t1b_ref):
    deg = jnp.sum(h_ref[...], axis=1, keepdims=True) + 1.0
    dinv = lax.rsqrt(deg)
    dinv_ref[...] = dinv
    t1 = dinv * jnp.dot(x_ref[...], w_ref[...],
                        preferred_element_type=jnp.float32)
    t1a_ref[...] = t1[:, :128]
    t1b_ref[...] = jnp.concatenate(
        [t1[:, 128:], jnp.zeros((t1.shape[0], 112), jnp.float32)], axis=1)


def _stageA(hist_t, xT, Wbd):
    return pl.pallas_call(
        _stageA_body,
        grid=(NBLK,),
        in_specs=[
            pl.BlockSpec((BLK, NW), lambda i: (i, 0)),
            pl.BlockSpec((BLK, 144), lambda i: (i, 0)),
            pl.BlockSpec((144, 144), lambda i: (0, 0)),
        ],
        out_specs=[
            pl.BlockSpec((BLK, 1), lambda i: (i, 0)),
            pl.BlockSpec((BLK, 128), lambda i: (i, 0)),
            pl.BlockSpec((BLK, 128), lambda i: (i, 0)),
        ],
        out_shape=[
            jax.ShapeDtypeStruct((NPAD, 1), jnp.float32),
            jax.ShapeDtypeStruct((NPAD, 128), jnp.float32),
            jax.ShapeDtypeStruct((NPAD, 128), jnp.float32),
        ],
    )(hist_t, xT, Wbd)


def _row_mask(nrows):
    i = pl.program_id(0)
    rows = i * BLK + lax.broadcasted_iota(jnp.int32, (nrows, 1), 0)
    return rows < N


def _stageB_body(a0a_ref, a1a_ref, t1a_ref, a0b_ref, a1b_ref, t1b_ref,
                 dinv_ref, w4_ref, b1_ref, o1l_ref, t2_ref):
    dinv = dinv_ref[...]
    fa = a0a_ref[...] + a1a_ref[...] - t1a_ref[...]
    fb = (a0b_ref[...] + a1b_ref[...] - t1b_ref[...])[:, :16]
    o1 = jnp.tanh(dinv * jnp.concatenate([fa, fb], axis=1) + b1_ref[...])
    o1 = jnp.where(_row_mask(o1.shape[0]), o1, 0.0)
    o1l_ref[...] = o1[:, :48]
    t2 = dinv * jnp.dot(o1[:, 48:], w4_ref[...],
                        preferred_element_type=jnp.float32)
    t2_ref[...] = jnp.concatenate(
        [t2, jnp.zeros((t2.shape[0], 32), jnp.float32)], axis=1)


def _stageB(a0a, a1a, T1a, a0b, a1b, T1b, dinv, Wbd4, bias1):
    blk128 = pl.BlockSpec((BLK, 128), lambda i: (i, 0))
    return pl.pallas_call(
        _stageB_body,
        grid=(NBLK,),
        in_specs=[
            blk128, blk128, blk128, blk128, blk128, blk128,
            pl.BlockSpec((BLK, 1), lambda i: (i, 0)),
            pl.BlockSpec((96, 96), lambda i: (0, 0)),
            pl.BlockSpec((1, 144), lambda i: (0, 0)),
        ],
        out_specs=[
            pl.BlockSpec((BLK, 48), lambda i: (i, 0)),
            pl.BlockSpec((BLK, 128), lambda i: (i, 0)),
        ],
        out_shape=[
            jax.ShapeDtypeStruct((NPAD, 48), jnp.float32),
            jax.ShapeDtypeStruct((NPAD, 128), jnp.float32),
        ],
    )(a0a, a1a, T1a, a0b, a1b, T1b, dinv, Wbd4, bias1)


def _stageC1_body(b0_ref, b1_ref, t2_ref, dinv_ref, b4_ref, o2_ref):
    f = (b0_ref[...] + b1_ref[...] - t2_ref[...])[:, :96]
    o2 = jnp.tanh(dinv_ref[...] * f + b4_ref[...])
    o2_ref[...] = jnp.where(_row_mask(o2.shape[0]), o2, 0.0)


def _stageC1(b0, b1_, T2, dinv, bias4):
    blk128 = pl.BlockSpec((BLK, 128), lambda i: (i, 0))
    return pl.pallas_call(
        _stageC1_body,
        grid=(NBLK,),
        in_specs=[
            blk128, blk128, blk128,
            pl.BlockSpec((BLK, 1), lambda i: (i, 0)),
            pl.BlockSpec((1, 96), lambda i: (0, 0)),
        ],
        out_specs=pl.BlockSpec((BLK, 96), lambda i: (i, 0)),
        out_shape=jax.ShapeDtypeStruct((NPAD, 96), jnp.float32),
    )(b0, b1_, T2, dinv, bias4)


NBLK2 = 26
KL = 29952 // NBLK2
KS = 59904 // NBLK2


def _stageC2_body(wl_ref, xl_ref, ws_ref, xs_ref, wlt_ref, xlt_ref,
                  wst_ref, xst_ref, bml_ref, bms_ref, wmu_ref,
                  mu_ref, accl, accs):
    i = pl.program_id(0)

    @pl.when(i == 0)
    def _():
        accl[...] = jnp.zeros_like(accl)
        accs[...] = jnp.zeros_like(accs)

    accl[...] += jnp.dot(wl_ref[...], xl_ref[...],
                         preferred_element_type=jnp.float32)
    accs[...] += jnp.dot(ws_ref[...], xs_ref[...],
                         preferred_element_type=jnp.float32)

    @pl.when(i == NBLK2 - 1)
    def _():
        tl = accl[...] + jnp.dot(wlt_ref[...], xlt_ref[...],
                                 preferred_element_type=jnp.float32)
        ts = accs[...] + jnp.dot(wst_ref[...], xst_ref[...],
                                 preferred_element_type=jnp.float32)
        netl = jnp.tanh(tl + bml_ref[...])
        nets = jnp.tanh(ts + bms_ref[...])
        net_t = jnp.concatenate([netl, nets], axis=0)
        mu_ref[...] = lax.dot_general(
            net_t, wmu_ref[...], (((0,), (1,)), ((), ())),
            preferred_element_type=jnp.float32)


def _stageC2(WmL, XrL, WmS, XrS, WmLt, WmSt, bmL, bmS, Wmu):
    return pl.pallas_call(
        _stageC2_body,
        grid=(NBLK2,),
        in_specs=[
            pl.BlockSpec((FEAT, KL), lambda i: (0, i)),
            pl.BlockSpec((KL, B), lambda i: (i, 0)),
            pl.BlockSpec((FEAT, KS), lambda i: (0, i)),
            pl.BlockSpec((KS, B), lambda i: (i, 0)),
            pl.BlockSpec((FEAT, 128), lambda i: (0, 0)),
            pl.BlockSpec((128, B), lambda i: (234, 0)),
            pl.BlockSpec((FEAT, 128), lambda i: (0, 0)),
            pl.BlockSpec((128, B), lambda i: (468, 0)),
            pl.BlockSpec((FEAT, 1), lambda i: (0, 0)),
            pl.BlockSpec((FEAT, 1), lambda i: (0, 0)),
            pl.BlockSpec((FEAT, 2 * FEAT), lambda i: (0, 0)),
        ],
        out_specs=pl.BlockSpec((B, FEAT), lambda i: (0, 0)),
        out_shape=jax.ShapeDtypeStruct((B, FEAT), jnp.float32),
        scratch_shapes=[
            pltpu.VMEM((FEAT, B), jnp.float32),
            pltpu.VMEM((FEAT, B), jnp.float32),
        ],
    )(WmL, XrL, WmS, XrS, WmLt, XrL, WmSt, XrS, bmL, bmS, Wmu)


def kernel(featurein, edge_index, W1_logr, b1_logr, W1_s, b1_s, W4_s, b4_s,
           Wm_logr, bm_logr, Wm_s, bm_s, Wmu):
    f32 = jnp.float32
    I16 = jnp.eye(B, dtype=f32)

    Wbd = jnp.zeros((144, 144), f32)
    Wbd = Wbd.at[:48, :48].set(jnp.kron(W1_logr / 4.0, I16))
    Wbd = Wbd.at[48:, 48:].set(jnp.kron(W1_s / 50.0, I16))
    Wbd4 = jnp.kron(W4_s, I16)
    bias1 = jnp.concatenate([jnp.repeat(b1_logr, B),
                             jnp.repeat(b1_s, B)]).reshape(1, 144)
    bias4 = jnp.repeat(b4_s, B).reshape(1, 96)

    xT = jnp.transpose(featurein, (1, 2, 0)).reshape(N, 9 * B)

    arp = jnp.arange(EPAD - E, dtype=jnp.int32)
    srcs = jnp.concatenate([edge_index[0], arp % N]).reshape(NCHUNK, CHUNK)
    dsts = jnp.concatenate([edge_index[1],
                            N + arp % (NPAD - N)]).reshape(NCHUNK, CHUNK)

    zeros1d = jnp.zeros((NPAD,), f32)
    hist = _run_deg(dsts, zeros1d)
    hist_t = hist.T

    dinv, T1a, T1b = _stageA(hist_t, xT, Wbd)

    a0a, a1a = _run_agg128(T1a, srcs, dsts)
    a0b, a1b = _run_agg128(T1b, srcs, dsts)
    o1logr, T2 = _stageB(a0a, a1a, T1a, a0b, a1b, T1b, dinv, Wbd4, bias1)

    b0, b1_ = _run_agg128(T2, srcs, dsts)
    o2 = _stageC1(b0, b1_, T2, dinv, bias4)

    XrL = o1logr.reshape(3 * NPAD, B)
    XrS = o2.reshape(6 * NPAD, B)
    WmLt = jnp.pad(Wm_logr[:, NBLK2 * KL:], ((0, 0), (0, 128 - 48)))
    WmSt = jnp.pad(Wm_s[:, NBLK2 * KS:], ((0, 0), (0, 128 - 96)))

    mu = _stageC2(Wm_logr, XrL, Wm_s, XrS, WmLt, WmSt,
                  bm_logr.reshape(FEAT, 1), bm_s.reshape(FEAT, 1), Wmu)
    return mu
